# baseline (speedup 1.0000x reference)
import functools

import jax
import jax.numpy as jnp
from jax import lax
from jax.experimental import pallas as pl
from jax.experimental.pallas import tpu as pltpu
from jax.experimental.pallas import tpu_sc as plsc

VOCAB = 1000000
EMBED_DIM = 64
BATCH = 16384
HIST_LEN = 50

NUM_IDX = BATCH * HIST_LEN
NUM_WORKERS = 32
PER_WORKER = NUM_IDX // NUM_WORKERS
CHUNK = 256
NCHUNK = PER_WORKER // CHUNK


NBUF = 5


def _gather_kernel(table, idx_hbm, out, idx_v, rows_v, gsems, osem):
    wid = lax.axis_index("s") * 2 + lax.axis_index("c")
    base = wid * PER_WORKER
    pltpu.sync_copy(idx_hbm.at[wid], idx_v)

    def start_gather(b, g):
        pltpu.async_copy(table.at[idx_v.at[g]], rows_v.at[b], gsems[b])

    for b in range(NBUF):
        start_gather(b, b)

    @pl.loop(0, NCHUNK, step=NBUF)
    def _body(g0):
        for b in range(NBUF):
            g = g0 + b
            pltpu.make_async_copy(table.at[idx_v.at[g]], rows_v.at[b],
                                  gsems[b]).wait()
            pltpu.async_copy(rows_v.at[b], out.at[pl.ds(base + g * CHUNK, CHUNK)],
                             osem)

            @pl.when(g + NBUF < NCHUNK)
            def _refill():
                pltpu.make_async_copy(
                    rows_v.at[b], out.at[pl.ds(base + g * CHUNK, CHUNK)],
                    osem).wait()
                start_gather(b, g + NBUF)

    for b in range(NBUF):
        g = NCHUNK - NBUF + b
        pltpu.make_async_copy(rows_v.at[b],
                              out.at[pl.ds(base + g * CHUNK, CHUNK)],
                              osem).wait()


def kernel(item_embeddings, batch_data):
    idx = batch_data.astype(jnp.int32).reshape(NUM_WORKERS, NCHUNK, CHUNK)
    mesh = plsc.VectorSubcoreMesh(core_axis_name="c", subcore_axis_name="s")
    flat = pl.kernel(
        _gather_kernel,
        out_type=jax.ShapeDtypeStruct((NUM_IDX, EMBED_DIM), jnp.float32),
        mesh=mesh,
        scratch_types=[
            pltpu.VMEM((NCHUNK, CHUNK), jnp.int32),
            pltpu.VMEM((NBUF, CHUNK, EMBED_DIM), jnp.float32),
            tuple(pltpu.SemaphoreType.DMA for _ in range(NBUF)),
            pltpu.SemaphoreType.DMA,
        ],
        compiler_params=pltpu.CompilerParams(use_tc_tiling_on_sc=False),
    )(item_embeddings, idx)
    return flat.reshape(BATCH, HIST_LEN, EMBED_DIM)


# speedup vs baseline: 1.0002x; 1.0002x over previous
import functools

import jax
import jax.numpy as jnp
from jax import lax
from jax.experimental import pallas as pl
from jax.experimental.pallas import tpu as pltpu
from jax.experimental.pallas import tpu_sc as plsc

VOCAB = 1000000
EMBED_DIM = 64
BATCH = 16384
HIST_LEN = 50

NUM_IDX = BATCH * HIST_LEN
NUM_WORKERS = 32
PER_WORKER = NUM_IDX // NUM_WORKERS
B_PER_CHUNK = 4
CHUNK = B_PER_CHUNK * HIST_LEN
NCHUNK = PER_WORKER // CHUNK
B_PER_WORKER = BATCH // NUM_WORKERS
NBUF = 8


def _gather_kernel(table, idx_hbm, out, idx_v, rows_v, gsems, osem):
    wid = lax.axis_index("s") * 2 + lax.axis_index("c")
    b_base = wid * B_PER_WORKER
    pltpu.sync_copy(idx_hbm.at[wid], idx_v)

    def start_out(b, g):
        for bb in range(B_PER_CHUNK):
            pltpu.async_copy(rows_v.at[b, pl.ds(bb * HIST_LEN, HIST_LEN)],
                             out.at[b_base + g * B_PER_CHUNK + bb], osem)

    def wait_out(b, g):
        for bb in range(B_PER_CHUNK):
            pltpu.make_async_copy(rows_v.at[b, pl.ds(bb * HIST_LEN, HIST_LEN)],
                                  out.at[b_base + g * B_PER_CHUNK + bb],
                                  osem).wait()

    def start_gather(b, g):
        pltpu.async_copy(table.at[idx_v.at[g]], rows_v.at[b], gsems[b])

    for b in range(NBUF):
        start_gather(b, b)

    @pl.loop(0, NCHUNK, step=NBUF)
    def _body(g0):
        for b in range(NBUF):
            g = g0 + b
            pltpu.make_async_copy(table.at[idx_v.at[g]], rows_v.at[b],
                                  gsems[b]).wait()
            start_out(b, g)

            @pl.when(g + NBUF < NCHUNK)
            def _refill():
                wait_out(b, g)
                start_gather(b, g + NBUF)

    for b in range(NBUF):
        wait_out(b, NCHUNK - NBUF + b)


def kernel(item_embeddings, batch_data):
    idx = batch_data.astype(jnp.int32).reshape(NUM_WORKERS, NCHUNK, CHUNK)
    mesh = plsc.VectorSubcoreMesh(core_axis_name="c", subcore_axis_name="s")
    return pl.kernel(
        _gather_kernel,
        out_type=jax.ShapeDtypeStruct((BATCH, HIST_LEN, EMBED_DIM),
                                      jnp.float32),
        mesh=mesh,
        scratch_types=[
            pltpu.VMEM((NCHUNK, CHUNK), jnp.int32),
            pltpu.VMEM((NBUF, CHUNK, EMBED_DIM), jnp.float32),
            tuple(pltpu.SemaphoreType.DMA for _ in range(NBUF)),
            pltpu.SemaphoreType.DMA,
        ],
        compiler_params=pltpu.CompilerParams(use_tc_tiling_on_sc=False),
    )(item_embeddings, idx)
